# baseline (speedup 1.0000x reference)
import functools

import jax
import jax.numpy as jnp
from jax import lax
from jax.experimental import pallas as pl
from jax.experimental.pallas import tpu as pltpu
from jax.experimental.pallas import tpu_sc as plsc

N = 10000
E = 320000
G = 256
DN = 128
DE = 32
H = 128
GOUT = 64
EPS = 1e-5

NC = 2
NS = 16
NT = NC * NS
PT = E // NT
C = 80
NCHUNK = PT // C
GRP = C // 16

NODE_CHUNK = 1000
EDGE_CHUNK = 3200

_BCAST_DNUMS = lax.GatherDimensionNumbers(
    offset_dims=(), collapsed_slice_dims=(0,), start_index_map=(0,))


def _bcast_lane(v16, lane):
    idx = jnp.full((16, 1), lane, jnp.int32)
    return lax.gather(v16, idx, _BCAST_DNUMS, (1,),
                      mode=lax.GatherScatterMode.PROMISE_IN_BOUNDS)


def _prep_body(x_ref, b3d_ref, w1_ref, b1_ref, xw_ref, gx_ref):
    i = pl.program_id(0)
    xc = x_ref[...]
    xw_ref[...] = xc @ w1_ref[:DN, :] + b1_ref[0, :][None, :]
    b = b3d_ref[0, 0, :]
    iota_g = lax.broadcasted_iota(jnp.int32, (NODE_CHUNK, G), 1)
    onehot = (b[:, None] == iota_g).astype(jnp.float32)
    part = lax.dot_general(onehot, xc, (((0,), (0,)), ((), ())),
                           preferred_element_type=jnp.float32)

    @pl.when(i == 0)
    def _():
        gx_ref[...] = part

    @pl.when(i > 0)
    def _():
        gx_ref[...] += part


def _ew_body(ea_ref, w1_ref, ew_ref):
    ew_ref[...] = ea_ref[...] @ w1_ref[DN:, :]


def _sc_body(xw_hbm, ew_hbm, row_hbm, col_hbm, batch_hbm,
             s_out, cnt_out, ss_out,
             batch_v, row_v, col_v, g_v, ew_v2, xw_v2,
             s_v, cnt_v, ss_v, sem):
    wid = lax.axis_index("s") * NC + lax.axis_index("c")
    tbase = wid * PT

    pltpu.sync_copy(batch_hbm, batch_v)
    pltpu.sync_copy(row_hbm.at[pl.ds(tbase, PT)], row_v)
    pltpu.sync_copy(col_hbm.at[pl.ds(tbase, PT)], col_v)

    zeros16 = jnp.zeros((16,), jnp.float32)
    iota16 = lax.broadcasted_iota(jnp.int32, (16,), 0)
    ones16 = jnp.ones((16,), jnp.float32)
    lane0 = iota16 == 0
    colj = [iota16 + j * 16 for j in range(H // 16)]

    @plsc.parallel_loop(0, G)
    def _(r):
        for j in range(H // 16):
            s_v[r, pl.ds(j * 16, 16)] = zeros16
    for j in range(G // 16):
        cnt_v[pl.ds(j * 16, 16)] = zeros16

    @plsc.parallel_loop(0, PT, 16, unroll=4)
    def _(i):
        g_v[pl.ds(i, 16)] = plsc.load_gather(batch_v, [col_v[pl.ds(i, 16)]])

    def _descs(c, bb):
        hb = tbase + c * C
        de = pltpu.make_async_copy(ew_hbm.at[pl.ds(hb, C)],
                                   ew_v2.at[pl.ds(bb * C, C)], sem.at[bb])
        dx = pltpu.make_async_copy(xw_hbm.at[row_v.at[pl.ds(c * C, C)]],
                                   xw_v2.at[pl.ds(bb * C, C)], sem.at[bb])
        return de, dx

    def _issue(c, bb):
        de, dx = _descs(c, bb)
        de.start()
        dx.start()

    for j in range(H // 16):
        ss_v[pl.ds(j * 16, 16)] = zeros16

    _issue(0, 0)
    _issue(1, 1)

    def _chunk(c, _):
        bb = c & 1
        de, dx = _descs(c, bb)
        de.wait()
        dx.wait()
        boff = bb * C

        @plsc.parallel_loop(0, C, 16, unroll=2)
        def _group(gi):
            g16 = g_v[pl.ds(c * C + gi, 16)]
            for e in range(16):
                ge = _bcast_lane(g16, e)
                erow = boff + gi + e
                for j in range(H // 16):
                    pre = xw_v2[erow, pl.ds(j * 16, 16)] + ew_v2[erow, pl.ds(j * 16, 16)]
                    a = jnp.maximum(pre, 0.01 * pre)
                    plsc.addupdate(ss_v.at[pl.ds(j * 16, 16)], a * a)
                    plsc.addupdate_scatter(s_v, [ge, colj[j]], a)
                plsc.addupdate_scatter(cnt_v, [ge], ones16, mask=lane0)

        @pl.when(c + 2 < NCHUNK)
        def _():
            _issue(c + 2, bb)
        return 0

    lax.fori_loop(0, NCHUNK, _chunk, 0)

    pltpu.sync_copy(s_v, s_out.at[wid])
    pltpu.sync_copy(cnt_v, cnt_out.at[wid])
    pltpu.sync_copy(ss_v, ss_out.at[wid])


def _final_body(s_ref, cnt_ref, ss_ref, gx_ref,
                g1_ref, be1_ref, w2_ref, b2_ref,
                w3_ref, b3_ref, g2_ref, be2_ref, w4_ref, b4_ref, out_ref):
    s = jnp.sum(s_ref[...], axis=0)
    cnt = jnp.sum(cnt_ref[...], axis=0)
    ss = jnp.sum(ss_ref[...], axis=0, keepdims=True)
    suma = jnp.sum(s, axis=0, keepdims=True)
    m = suma / E
    v = ss / E - m * m
    alpha = g1_ref[...] * lax.rsqrt(v + EPS)
    beta = be1_ref[...] - m * alpha
    go = (s * alpha + cnt * beta) @ w2_ref[...] + cnt * b2_ref[...]
    h2 = gx_ref[...] @ w3_ref[:DN, :] + go @ w3_ref[DN:, :] + b3_ref[...]
    h2 = jnp.maximum(h2, 0.01 * h2)
    m2 = jnp.mean(h2, axis=0, keepdims=True)
    d2 = h2 - m2
    v2 = jnp.mean(d2 * d2, axis=0, keepdims=True)
    hn = d2 * lax.rsqrt(v2 + EPS) * g2_ref[...] + be2_ref[...]
    out_ref[...] = hn @ w4_ref[...] + b4_ref[...]


def kernel(x, edge_index, edge_attr, u, batch,
           W1, b1, g1, be1, W2, b2,
           W3, b3, g2, be2, W4, b4):
    del u
    batch3 = batch.reshape(N // NODE_CHUNK, 1, NODE_CHUNK)
    b1_2 = jnp.broadcast_to(b1.reshape(1, H), (8, H))

    xw, gx = pl.pallas_call(
        _prep_body,
        grid=(N // NODE_CHUNK,),
        in_specs=[
            pl.BlockSpec((NODE_CHUNK, DN), lambda i: (i, 0)),
            pl.BlockSpec((1, 1, NODE_CHUNK), lambda i: (i, 0, 0)),
            pl.BlockSpec((DN + DE, H), lambda i: (0, 0)),
            pl.BlockSpec((8, H), lambda i: (0, 0)),
        ],
        out_specs=[
            pl.BlockSpec((NODE_CHUNK, DN), lambda i: (i, 0)),
            pl.BlockSpec((G, DN), lambda i: (0, 0)),
        ],
        out_shape=[
            jax.ShapeDtypeStruct((N, DN), jnp.float32),
            jax.ShapeDtypeStruct((G, DN), jnp.float32),
        ],
    )(x, batch3, W1, b1_2)

    ew = pl.pallas_call(
        _ew_body,
        grid=(E // EDGE_CHUNK,),
        in_specs=[
            pl.BlockSpec((EDGE_CHUNK, DE), lambda i: (i, 0)),
            pl.BlockSpec((DN + DE, H), lambda i: (0, 0)),
        ],
        out_specs=pl.BlockSpec((EDGE_CHUNK, H), lambda i: (i, 0)),
        out_shape=jax.ShapeDtypeStruct((E, H), jnp.float32),
    )(edge_attr, W1)

    mesh = plsc.VectorSubcoreMesh(core_axis_name="c", subcore_axis_name="s")
    s_parts, cnt_parts, ss_parts = pl.kernel(
        _sc_body,
        out_type=[
            jax.ShapeDtypeStruct((NT, G, H), jnp.float32),
            jax.ShapeDtypeStruct((NT, G), jnp.float32),
            jax.ShapeDtypeStruct((NT, H), jnp.float32),
        ],
        mesh=mesh,
        compiler_params=pltpu.CompilerParams(needs_layout_passes=False),
        scratch_types=[
            pltpu.VMEM((N,), jnp.int32),
            pltpu.VMEM((PT,), jnp.int32),
            pltpu.VMEM((PT,), jnp.int32),
            pltpu.VMEM((PT,), jnp.int32),
            pltpu.VMEM((2 * C, H), jnp.float32),
            pltpu.VMEM((2 * C, H), jnp.float32),
            pltpu.VMEM((G, H), jnp.float32),
            pltpu.VMEM((G,), jnp.float32),
            pltpu.VMEM((H,), jnp.float32),
            pltpu.SemaphoreType.DMA((2,)),
        ],
    )(xw, ew, edge_index[0], edge_index[1], batch)

    out = pl.pallas_call(
        _final_body,
        out_shape=jax.ShapeDtypeStruct((G, GOUT), jnp.float32),
    )(s_parts, cnt_parts.reshape(NT, G, 1), ss_parts, gx,
      g1.reshape(1, H), be1.reshape(1, H), W2, b2.reshape(1, H),
      W3, b3.reshape(1, H), g2.reshape(1, H), be2.reshape(1, H),
      W4, b4.reshape(1, GOUT))
    return out


# speedup vs baseline: 3.3017x; 3.3017x over previous
import functools

import jax
import jax.numpy as jnp
from jax import lax
from jax.experimental import pallas as pl
from jax.experimental.pallas import tpu as pltpu
from jax.experimental.pallas import tpu_sc as plsc

N = 10000
E = 320000
G = 256
DN = 128
DE = 32
H = 128
GOUT = 64
EPS = 1e-5

NC = 2
NS = 16
NT = NC * NS
PT = E // NT
C = 80
NCHUNK = PT // C
GRP = C // 16

NODE_CHUNK = 1000
EDGE_CHUNK = 3200

_BCAST_DNUMS = lax.GatherDimensionNumbers(
    offset_dims=(), collapsed_slice_dims=(0,), start_index_map=(0,))


def _bcast_lane(v16, lane):
    idx = jnp.full((16, 1), lane, jnp.int32)
    return lax.gather(v16, idx, _BCAST_DNUMS, (1,),
                      mode=lax.GatherScatterMode.PROMISE_IN_BOUNDS)


def _prep_body(x_ref, b3d_ref, w1_ref, b1_ref, xw_ref, gx_ref):
    i = pl.program_id(0)
    xc = x_ref[...]
    xw_ref[...] = xc @ w1_ref[:DN, :] + b1_ref[0, :][None, :]
    b = b3d_ref[0, 0, :]
    iota_g = lax.broadcasted_iota(jnp.int32, (NODE_CHUNK, G), 1)
    onehot = (b[:, None] == iota_g).astype(jnp.float32)
    part = lax.dot_general(onehot, xc, (((0,), (0,)), ((), ())),
                           preferred_element_type=jnp.float32)

    @pl.when(i == 0)
    def _():
        gx_ref[...] = part

    @pl.when(i > 0)
    def _():
        gx_ref[...] += part


def _ew_body(ea_ref, w1_ref, ew_ref):
    ew_ref[...] = ea_ref[...] @ w1_ref[DN:, :]


def _sc_body(xw_hbm, ew_hbm, row_hbm, col_hbm, batch_hbm,
             s_out, cnt_out, ss_out,
             batch_v, row_v, col_v, g_v, ew_v2, xw_v2,
             s_v, cnt_v, ss_v, sem):
    wid = lax.axis_index("s") * NC + lax.axis_index("c")
    tbase = wid * PT

    pltpu.sync_copy(batch_hbm, batch_v)
    pltpu.sync_copy(row_hbm.at[pl.ds(tbase, PT)], row_v)
    pltpu.sync_copy(col_hbm.at[pl.ds(tbase, PT)], col_v)

    zeros16 = jnp.zeros((16,), jnp.float32)
    iota16 = lax.broadcasted_iota(jnp.int32, (16,), 0)
    ones16 = jnp.ones((16,), jnp.float32)
    lane0 = iota16 == 0
    colj = [iota16 + j * 16 for j in range(H // 16)]

    @plsc.parallel_loop(0, G)
    def _(r):
        for j in range(H // 16):
            s_v[r, pl.ds(j * 16, 16)] = zeros16
    for j in range(G // 16):
        cnt_v[pl.ds(j * 16, 16)] = zeros16

    @plsc.parallel_loop(0, PT, 16, unroll=4)
    def _(i):
        g_v[pl.ds(i, 16)] = plsc.load_gather(batch_v, [col_v[pl.ds(i, 16)]])

    def _descs(c, bb):
        hb = tbase + c * C
        de = pltpu.make_async_copy(ew_hbm.at[pl.ds(hb, C)],
                                   ew_v2.at[pl.ds(bb * C, C)], sem.at[bb])
        dx = pltpu.make_async_copy(xw_hbm.at[row_v.at[pl.ds(c * C, C)]],
                                   xw_v2.at[pl.ds(bb * C, C)], sem.at[bb])
        return de, dx

    def _issue(c, bb):
        de, dx = _descs(c, bb)
        de.start()
        dx.start()

    for j in range(H // 16):
        ss_v[pl.ds(j * 16, 16)] = zeros16

    _issue(0, 0)
    _issue(1, 1)

    NJ = H // 16

    def _chunk(c, _):
        bb = c & 1
        de, dx = _descs(c, bb)
        de.wait()
        dx.wait()
        boff = bb * C

        def _group(gi, _):
            g16 = g_v[pl.ds(c * C + gi, 16)]
            row0 = boff + gi
            xwc = [xw_v2[row0, pl.ds(j * 16, 16)] for j in range(NJ)]
            ewc = [ew_v2[row0, pl.ds(j * 16, 16)] for j in range(NJ)]
            for e in range(16):
                ge = _bcast_lane(g16, e)
                xwn, ewn = [None] * NJ, [None] * NJ
                for j in range(NJ):
                    if e + 1 < 16:
                        xwn[j] = xw_v2[row0 + e + 1, pl.ds(j * 16, 16)]
                        ewn[j] = ew_v2[row0 + e + 1, pl.ds(j * 16, 16)]
                    pre = xwc[j] + ewc[j]
                    a = jnp.maximum(pre, 0.01 * pre)
                    plsc.addupdate(ss_v.at[pl.ds(j * 16, 16)], a * a)
                    plsc.addupdate_scatter(s_v, [ge, colj[j]], a)
                plsc.addupdate_scatter(cnt_v, [ge], ones16, mask=lane0)
                xwc, ewc = xwn, ewn
            return 0

        lax.fori_loop(0, GRP, lambda t, _: _group(t * 16, 0), 0)

        @pl.when(c + 2 < NCHUNK)
        def _():
            _issue(c + 2, bb)
        return 0

    lax.fori_loop(0, NCHUNK, _chunk, 0)

    pltpu.sync_copy(s_v, s_out.at[wid])
    pltpu.sync_copy(cnt_v, cnt_out.at[wid])
    pltpu.sync_copy(ss_v, ss_out.at[wid])


def _final_body(s_ref, cnt_ref, ss_ref, gx_ref,
                g1_ref, be1_ref, w2_ref, b2_ref,
                w3_ref, b3_ref, g2_ref, be2_ref, w4_ref, b4_ref, out_ref):
    s = jnp.sum(s_ref[...], axis=0)
    cnt = jnp.sum(cnt_ref[...], axis=0)
    ss = jnp.sum(ss_ref[...], axis=0, keepdims=True)
    suma = jnp.sum(s, axis=0, keepdims=True)
    m = suma / E
    v = ss / E - m * m
    alpha = g1_ref[...] * lax.rsqrt(v + EPS)
    beta = be1_ref[...] - m * alpha
    go = (s * alpha + cnt * beta) @ w2_ref[...] + cnt * b2_ref[...]
    h2 = gx_ref[...] @ w3_ref[:DN, :] + go @ w3_ref[DN:, :] + b3_ref[...]
    h2 = jnp.maximum(h2, 0.01 * h2)
    m2 = jnp.mean(h2, axis=0, keepdims=True)
    d2 = h2 - m2
    v2 = jnp.mean(d2 * d2, axis=0, keepdims=True)
    hn = d2 * lax.rsqrt(v2 + EPS) * g2_ref[...] + be2_ref[...]
    out_ref[...] = hn @ w4_ref[...] + b4_ref[...]


def kernel(x, edge_index, edge_attr, u, batch,
           W1, b1, g1, be1, W2, b2,
           W3, b3, g2, be2, W4, b4):
    del u
    batch3 = batch.reshape(N // NODE_CHUNK, 1, NODE_CHUNK)
    b1_2 = jnp.broadcast_to(b1.reshape(1, H), (8, H))

    xw, gx = pl.pallas_call(
        _prep_body,
        grid=(N // NODE_CHUNK,),
        in_specs=[
            pl.BlockSpec((NODE_CHUNK, DN), lambda i: (i, 0)),
            pl.BlockSpec((1, 1, NODE_CHUNK), lambda i: (i, 0, 0)),
            pl.BlockSpec((DN + DE, H), lambda i: (0, 0)),
            pl.BlockSpec((8, H), lambda i: (0, 0)),
        ],
        out_specs=[
            pl.BlockSpec((NODE_CHUNK, DN), lambda i: (i, 0)),
            pl.BlockSpec((G, DN), lambda i: (0, 0)),
        ],
        out_shape=[
            jax.ShapeDtypeStruct((N, DN), jnp.float32),
            jax.ShapeDtypeStruct((G, DN), jnp.float32),
        ],
    )(x, batch3, W1, b1_2)

    ew = pl.pallas_call(
        _ew_body,
        grid=(E // EDGE_CHUNK,),
        in_specs=[
            pl.BlockSpec((EDGE_CHUNK, DE), lambda i: (i, 0)),
            pl.BlockSpec((DN + DE, H), lambda i: (0, 0)),
        ],
        out_specs=pl.BlockSpec((EDGE_CHUNK, H), lambda i: (i, 0)),
        out_shape=jax.ShapeDtypeStruct((E, H), jnp.float32),
    )(edge_attr, W1)

    mesh = plsc.VectorSubcoreMesh(core_axis_name="c", subcore_axis_name="s")
    s_parts, cnt_parts, ss_parts = pl.kernel(
        _sc_body,
        out_type=[
            jax.ShapeDtypeStruct((NT, G, H), jnp.float32),
            jax.ShapeDtypeStruct((NT, G), jnp.float32),
            jax.ShapeDtypeStruct((NT, H), jnp.float32),
        ],
        mesh=mesh,
        compiler_params=pltpu.CompilerParams(needs_layout_passes=False),
        scratch_types=[
            pltpu.VMEM((N,), jnp.int32),
            pltpu.VMEM((PT,), jnp.int32),
            pltpu.VMEM((PT,), jnp.int32),
            pltpu.VMEM((PT,), jnp.int32),
            pltpu.VMEM((2 * C, H), jnp.float32),
            pltpu.VMEM((2 * C, H), jnp.float32),
            pltpu.VMEM((G, H), jnp.float32),
            pltpu.VMEM((G,), jnp.float32),
            pltpu.VMEM((H,), jnp.float32),
            pltpu.SemaphoreType.DMA((2,)),
        ],
    )(xw, ew, edge_index[0], edge_index[1], batch)

    out = pl.pallas_call(
        _final_body,
        out_shape=jax.ShapeDtypeStruct((G, GOUT), jnp.float32),
    )(s_parts, cnt_parts.reshape(NT, G, 1), ss_parts, gx,
      g1.reshape(1, H), be1.reshape(1, H), W2, b2.reshape(1, H),
      W3, b3.reshape(1, H), g2.reshape(1, H), be2.reshape(1, H),
      W4, b4.reshape(1, GOUT))
    return out
